# baseline (speedup 1.0000x reference)
import jax
import jax.numpy as jnp
from jax.experimental import pallas as pl
from jax.experimental.pallas import tpu as pltpu

_DEPTH = 4
_K = 1024
_D = 256


def _rq_kernel(x_ref, cbf_ref, cbsq_ref, cbs_ref, out_ref, codes_ref,
               loss_ref):
    step = pl.program_id(0)

    @pl.when(step == 0)
    def _():
        loss_ref[...] = jnp.zeros((1, 1), jnp.float32)

    T = x_ref.shape[0]
    H = T // 2
    loss_acc = jnp.zeros((), jnp.float32)
    lane = jax.lax.broadcasted_iota(jnp.int32, (H, _K), 1)
    for h in range(2):
        xb = x_ref[h * H:(h + 1) * H, :]
        residual = xb
        agg = jnp.zeros_like(xb)
        code_cols = []
        for i in range(_DEPTH):
            norm = jnp.sqrt(
                jnp.sum(residual * residual, axis=1, keepdims=True))
            inp = residual / jnp.maximum(norm, 1e-12)
            in_sq = jnp.sum(inp * inp, axis=1, keepdims=True)
            inp_bf = inp.astype(jnp.bfloat16)

            ab = jax.lax.dot_general(
                inp_bf, cbf_ref[i], (((1,), (1,)), ((), ())),
                preferred_element_type=jnp.float32)
            scores = in_sq + cbsq_ref[i] - 2.0 * ab
            best_idx = jnp.argmin(scores, axis=1)[:, None]

            onehot = (lane == best_idx).astype(jnp.bfloat16)
            q3 = jax.lax.dot_general(
                onehot, cbs_ref[i], (((1,), (0,)), ((), ())),
                preferred_element_type=jnp.float32)
            quant = (q3[:, :_D] + q3[:, _D:2 * _D]) + q3[:, 2 * _D:]

            residual = residual - quant
            agg = agg + quant
            diff = xb - agg
            loss_acc = loss_acc + jnp.sum(diff * diff)
            code_cols.append(best_idx)

        out_ref[h * H:(h + 1) * H, :] = xb + (agg - xb)
        codes_ref[h * H:(h + 1) * H, :] = jnp.concatenate(code_cols, axis=1)

    loss_ref[...] += jnp.reshape(loss_acc, (1, 1))


@jax.jit
def kernel(x, codebooks):
    orig_shape = x.shape
    N = x.shape[0] * x.shape[1] * x.shape[2]
    D = x.shape[3]
    flat = x.reshape(N, D)

    cb_bf = codebooks.astype(jnp.bfloat16)
    cb_sq = jnp.stack([jnp.sum(codebooks[i] * codebooks[i], axis=1)
                       for i in range(_DEPTH)])[:, None, :]

    mask = jnp.uint32(0xFFFF0000)

    def trunc_bf16(v):
        u = jax.lax.bitcast_convert_type(v, jnp.uint32)
        return jax.lax.bitcast_convert_type(u & mask, jnp.float32)

    hi_f = trunc_bf16(codebooks)
    r1 = codebooks - hi_f
    mid_f = trunc_bf16(r1)
    r2 = r1 - mid_f
    lo_f = trunc_bf16(r2)
    cb_split = jnp.concatenate(
        [hi_f.astype(jnp.bfloat16), mid_f.astype(jnp.bfloat16),
         lo_f.astype(jnp.bfloat16)], axis=2)

    T = 512
    grid = (N // T,)

    out, codes, loss = pl.pallas_call(
        _rq_kernel,
        grid=grid,
        in_specs=[
            pl.BlockSpec((T, D), lambda i: (i, 0)),
            pl.BlockSpec((_DEPTH, _K, D), lambda i: (0, 0, 0)),
            pl.BlockSpec((_DEPTH, 1, _K), lambda i: (0, 0, 0)),
            pl.BlockSpec((_DEPTH, _K, 3 * D), lambda i: (0, 0, 0)),
        ],
        out_specs=[
            pl.BlockSpec((T, D), lambda i: (i, 0)),
            pl.BlockSpec((T, _DEPTH), lambda i: (i, 0)),
            pl.BlockSpec((1, 1), lambda i: (0, 0)),
        ],
        out_shape=[
            jax.ShapeDtypeStruct((N, D), jnp.float32),
            jax.ShapeDtypeStruct((N, _DEPTH), jnp.int32),
            jax.ShapeDtypeStruct((1, 1), jnp.float32),
        ],
    )(flat, cb_bf, cb_sq, cb_split)

    quants = out.reshape(orig_shape)
    codes = codes.reshape(orig_shape[:-1] + (_DEPTH,))
    commitment_loss = loss[0, 0] / (N * D * _DEPTH)
    return quants, commitment_loss, codes


# speedup vs baseline: 1.0946x; 1.0946x over previous
import jax
import jax.numpy as jnp
from jax.experimental import pallas as pl
from jax.experimental.pallas import tpu as pltpu

_DEPTH = 4
_K = 1024
_D = 256


def _rq_kernel(x_ref, cbf_ref, cbsq_ref, cbs_ref, out_ref, codes_ref,
               loss_ref):
    step = pl.program_id(0)

    @pl.when(step == 0)
    def _():
        loss_ref[...] = jnp.zeros((1, 1), jnp.float32)

    T = x_ref.shape[0]
    H = T // 2
    loss_acc = jnp.zeros((), jnp.float32)
    lane = jax.lax.broadcasted_iota(jnp.int32, (H, _K), 1)
    for h in range(2):
        xb = x_ref[h * H:(h + 1) * H, :]
        residual = xb
        agg = jnp.zeros_like(xb)
        code_cols = []
        for i in range(_DEPTH):
            norm = jnp.sqrt(
                jnp.sum(residual * residual, axis=1, keepdims=True))
            inp = residual / jnp.maximum(norm, 1e-12)
            in_sq = jnp.sum(inp * inp, axis=1, keepdims=True)
            inp_bf = inp.astype(jnp.bfloat16)

            ab = jax.lax.dot_general(
                inp_bf, cbf_ref[i], (((1,), (1,)), ((), ())),
                preferred_element_type=jnp.float32)
            scores = in_sq + cbsq_ref[i] - 2.0 * ab
            best_idx = jnp.argmin(scores, axis=1)[:, None]

            onehot = (lane == best_idx).astype(jnp.bfloat16)
            q3 = jax.lax.dot_general(
                onehot, cbs_ref[i], (((1,), (0,)), ((), ())),
                preferred_element_type=jnp.float32)
            quant = (q3[:, :_D] + q3[:, _D:2 * _D]) + q3[:, 2 * _D:]

            residual = residual - quant
            agg = agg + quant
            diff = xb - agg
            loss_acc = loss_acc + jnp.sum(diff * diff)
            code_cols.append(best_idx)

        out_ref[h * H:(h + 1) * H, :] = xb + (agg - xb)
        codes_ref[h * H:(h + 1) * H, :] = jnp.concatenate(code_cols, axis=1)

    loss_ref[...] += jnp.reshape(loss_acc, (1, 1))


@jax.jit
def kernel(x, codebooks):
    orig_shape = x.shape
    N = x.shape[0] * x.shape[1] * x.shape[2]
    D = x.shape[3]
    flat = x.reshape(N, D)

    cb_bf = codebooks.astype(jnp.bfloat16)
    cb_sq = jnp.stack([jnp.sum(codebooks[i] * codebooks[i], axis=1)
                       for i in range(_DEPTH)])[:, None, :]

    mask = jnp.uint32(0xFFFF0000)

    def trunc_bf16(v):
        u = jax.lax.bitcast_convert_type(v, jnp.uint32)
        return jax.lax.bitcast_convert_type(u & mask, jnp.float32)

    hi_f = trunc_bf16(codebooks)
    r1 = codebooks - hi_f
    mid_f = trunc_bf16(r1)
    r2 = r1 - mid_f
    lo_f = trunc_bf16(r2)
    cb_split = jnp.concatenate(
        [hi_f.astype(jnp.bfloat16), mid_f.astype(jnp.bfloat16),
         lo_f.astype(jnp.bfloat16)], axis=2)

    T = 1024
    grid = (N // T,)

    out, codes, loss = pl.pallas_call(
        _rq_kernel,
        grid=grid,
        in_specs=[
            pl.BlockSpec((T, D), lambda i: (i, 0)),
            pl.BlockSpec((_DEPTH, _K, D), lambda i: (0, 0, 0)),
            pl.BlockSpec((_DEPTH, 1, _K), lambda i: (0, 0, 0)),
            pl.BlockSpec((_DEPTH, _K, 3 * D), lambda i: (0, 0, 0)),
        ],
        out_specs=[
            pl.BlockSpec((T, D), lambda i: (i, 0)),
            pl.BlockSpec((T, _DEPTH), lambda i: (i, 0)),
            pl.BlockSpec((1, 1), lambda i: (0, 0)),
        ],
        out_shape=[
            jax.ShapeDtypeStruct((N, D), jnp.float32),
            jax.ShapeDtypeStruct((N, _DEPTH), jnp.int32),
            jax.ShapeDtypeStruct((1, 1), jnp.float32),
        ],
    )(flat, cb_bf, cb_sq, cb_split)

    quants = out.reshape(orig_shape)
    codes = codes.reshape(orig_shape[:-1] + (_DEPTH,))
    commitment_loss = loss[0, 0] / (N * D * _DEPTH)
    return quants, commitment_loss, codes


# speedup vs baseline: 1.1092x; 1.0134x over previous
import jax
import jax.numpy as jnp
from jax.experimental import pallas as pl
from jax.experimental.pallas import tpu as pltpu

_DEPTH = 4
_K = 1024
_D = 256


def _rq_kernel(x_ref, cbf_ref, cbsq_ref, cbs_ref, out_ref, codes_ref,
               loss_ref):
    step = pl.program_id(0)

    @pl.when(step == 0)
    def _():
        loss_ref[...] = jnp.zeros((1, 1), jnp.float32)

    T = x_ref.shape[0]
    H = T // 2
    loss_acc = jnp.zeros((), jnp.float32)
    lane = jax.lax.broadcasted_iota(jnp.int32, (H, _K), 1)
    for h in range(2):
        xb = x_ref[h * H:(h + 1) * H, :]
        residual = xb
        agg = jnp.zeros_like(xb)
        code_cols = []
        for i in range(_DEPTH):
            norm = jnp.sqrt(
                jnp.sum(residual * residual, axis=1, keepdims=True))
            inp = residual / jnp.maximum(norm, 1e-12)
            in_sq = jnp.sum(inp * inp, axis=1, keepdims=True)
            inp_bf = inp.astype(jnp.bfloat16)

            ab = jax.lax.dot_general(
                inp_bf, cbf_ref[i], (((1,), (1,)), ((), ())),
                preferred_element_type=jnp.float32)
            scores = in_sq + cbsq_ref[i] - 2.0 * ab
            best_idx = jnp.argmin(scores, axis=1)[:, None]

            onehot = (lane == best_idx).astype(jnp.bfloat16)
            q3 = jax.lax.dot_general(
                onehot, cbs_ref[i], (((1,), (0,)), ((), ())),
                preferred_element_type=jnp.float32)
            quant = (q3[:, :_D] + q3[:, _D:2 * _D]) + q3[:, 2 * _D:]

            residual = residual - quant
            agg = agg + quant
            diff = xb - agg
            loss_acc = loss_acc + jnp.sum(diff * diff)
            code_cols.append(best_idx)

        out_ref[h * H:(h + 1) * H, :] = xb + (agg - xb)
        codes_ref[h * H:(h + 1) * H, :] = jnp.concatenate(code_cols, axis=1)

    loss_ref[...] += jnp.reshape(loss_acc, (1, 1))


@jax.jit
def kernel(x, codebooks):
    orig_shape = x.shape
    N = x.shape[0] * x.shape[1] * x.shape[2]
    D = x.shape[3]
    flat = x.reshape(N, D)

    cb_bf = codebooks.astype(jnp.bfloat16)
    cb_sq = jnp.stack([jnp.sum(codebooks[i] * codebooks[i], axis=1)
                       for i in range(_DEPTH)])[:, None, :]

    mask = jnp.uint32(0xFFFF0000)

    def trunc_bf16(v):
        u = jax.lax.bitcast_convert_type(v, jnp.uint32)
        return jax.lax.bitcast_convert_type(u & mask, jnp.float32)

    hi_f = trunc_bf16(codebooks)
    r1 = codebooks - hi_f
    mid_f = trunc_bf16(r1)
    r2 = r1 - mid_f
    lo_f = trunc_bf16(r2)
    cb_split = jnp.concatenate(
        [hi_f.astype(jnp.bfloat16), mid_f.astype(jnp.bfloat16),
         lo_f.astype(jnp.bfloat16)], axis=2)

    T = 2304
    grid = (N // T,)

    out, codes, loss = pl.pallas_call(
        _rq_kernel,
        grid=grid,
        in_specs=[
            pl.BlockSpec((T, D), lambda i: (i, 0)),
            pl.BlockSpec((_DEPTH, _K, D), lambda i: (0, 0, 0)),
            pl.BlockSpec((_DEPTH, 1, _K), lambda i: (0, 0, 0)),
            pl.BlockSpec((_DEPTH, _K, 3 * D), lambda i: (0, 0, 0)),
        ],
        out_specs=[
            pl.BlockSpec((T, D), lambda i: (i, 0)),
            pl.BlockSpec((T, _DEPTH), lambda i: (i, 0)),
            pl.BlockSpec((1, 1), lambda i: (0, 0)),
        ],
        out_shape=[
            jax.ShapeDtypeStruct((N, D), jnp.float32),
            jax.ShapeDtypeStruct((N, _DEPTH), jnp.int32),
            jax.ShapeDtypeStruct((1, 1), jnp.float32),
        ],
    )(flat, cb_bf, cb_sq, cb_split)

    quants = out.reshape(orig_shape)
    codes = codes.reshape(orig_shape[:-1] + (_DEPTH,))
    commitment_loss = loss[0, 0] / (N * D * _DEPTH)
    return quants, commitment_loss, codes
